# baseline (speedup 1.0000x reference)
import functools

import jax
import jax.numpy as jnp
from jax import lax
from jax.experimental import pallas as pl
from jax.experimental.pallas import tpu as pltpu
from jax.experimental.pallas import tpu_sc as plsc

SEQ = 200
BATCH = 4096
HIDDEN = 64
WIDE = 2 * HIDDEN
VOCAB = 1000000
CHUNK = 128
NC = 2
NS = 16
NW = NC * NS
CPW = SEQ
NBUF = 4
LOOK = 2

RCH = 256
RMAIN = VOCAB - HIDDEN
RNCH = RMAIN // RCH
RVIS = 2 * ((RNCH // NW + 2) // 2)


def _repack_body(tabT_hbm, tail_hbm, t2_hbm,
                 tb0, tb1, sb0, sb1, ttb, tsb, s0, s1, w0, w1, ts0):
    tbufs = (tb0, tb1)
    sbufs = (sb0, sb1)
    rsems = (s0, s1)
    wsems = (w0, w1)
    wid = lax.axis_index("s") * NC + lax.axis_index("c")
    lanes = jnp.arange(16, dtype=jnp.int32)

    def rd(k, b):
        cid = wid + k * NW
        return pltpu.make_async_copy(
            tabT_hbm.at[:, pl.ds(cid * RCH, RCH)], tbufs[b], rsems[b])

    def wr(k, b):
        cid = wid + k * NW
        return pltpu.make_async_copy(
            sbufs[b], t2_hbm.at[pl.ds(cid * RCH, RCH)], wsems[b])

    def valid(k):
        return wid + k * NW < RNCH

    def transpose(b):
        tb, sb = tbufs[b], sbufs[b]

        def rows(r, carry):
            rv = jnp.full((16,), r, dtype=jnp.int32)
            for j in range(4):
                sb[r, pl.ds(j * 16, 16)] = plsc.load_gather(
                    tb, [j * 16 + lanes, rv])
            return carry

        lax.fori_loop(0, RCH, rows, 0)

    rd(0, 0).start()

    def group(g, carry):
        for b in range(2):
            k = g * 2 + b
            nb = b ^ 1

            @pl.when(jnp.logical_and(k >= 1, valid(k - 1)))
            def _():
                wr(k - 1, nb).wait()

            @pl.when(valid(k + 1))
            def _():
                rd(k + 1, nb).start()

            @pl.when(valid(k))
            def _():
                rd(k, b).wait()
                transpose(b)
                wr(k, b).start()
        return carry

    lax.fori_loop(0, RVIS // 2, group, 0)

    @pl.when(valid(RVIS - 1))
    def _():
        wr(RVIS - 1, (RVIS - 1) % 2).wait()

    @pl.when(wid == 0)
    def _():
        pltpu.sync_copy(tail_hbm, ttb)
        for r4 in range(16):
            for rr in range(4):
                r = r4 * 4 + rr
                for j in range(4):
                    tsb[r, pl.ds(j * 16, 16)] = ttb[r, pl.ds(j * 16, 16)]
        pltpu.make_async_copy(
            tsb, t2_hbm.at[pl.ds(RMAIN, HIDDEN)], ts0).start()
        pltpu.make_async_copy(
            tsb, t2_hbm.at[pl.ds(RMAIN, HIDDEN)], ts0).wait()


def _gather_body(idx_hbm, t2_hbm, out_hbm, idx_v, *rest):
    gbufs = rest[:NBUF]
    tbufs = rest[NBUF:NBUF + 2]
    sems = rest[NBUF + 2:2 * NBUF + 2]
    stsems = rest[2 * NBUF + 2:]
    wid = lax.axis_index("s") * NC + lax.axis_index("c")
    col0 = wid * CHUNK
    lanes = jnp.arange(16, dtype=jnp.int32)

    def out_at(c):
        return out_hbm.at[c, :, pl.ds(col0, CHUNK)]

    def gather(c, b):
        pltpu.make_async_copy(t2_hbm.at[idx_v.at[c]], gbufs[b], sems[b]).start()

    def store(c, tb):
        return pltpu.make_async_copy(tbufs[tb], out_at(c), stsems[tb])

    def transpose(b, tb):
        gb, ob = gbufs[b], tbufs[tb]

        def rows(h, carry):
            hv = jnp.full((16,), h, dtype=jnp.int32)
            for j in range(8):
                ob[h, pl.ds(j * 16, 16)] = plsc.load_gather(
                    gb, [j * 16 + lanes, hv])
            return carry

        lax.fori_loop(0, HIDDEN, rows, 0)

    pltpu.sync_copy(idx_hbm.at[:, pl.ds(col0, CHUNK)], idx_v)

    for c in range(LOOK):
        gather(c, c % NBUF)

    def group(g, carry):
        for b in range(NBUF):
            c = g * NBUF + b
            pb = (b + LOOK) % NBUF
            tb = b % 2

            @pl.when(c + LOOK < CPW)
            def _():
                gather(c + LOOK, pb)

            pltpu.make_async_copy(t2_hbm.at[idx_v.at[c]], gbufs[b], sems[b]).wait()

            @pl.when(c >= 2)
            def _():
                store(c - 2, tb).wait()

            transpose(b, tb)
            store(c, tb).start()
        return carry

    lax.fori_loop(0, CPW // NBUF, group, 0)

    for c in range(CPW - 2, CPW):
        store(c, c % 2).wait()


def kernel(input_ids, table):
    mesh = plsc.VectorSubcoreMesh(core_axis_name="c", subcore_axis_name="s")
    tail = lax.slice(table, (RMAIN, 0), (VOCAB, HIDDEN))
    repack = functools.partial(
        pl.kernel,
        mesh=mesh,
        compiler_params=pltpu.CompilerParams(needs_layout_passes=False),
        out_type=jax.ShapeDtypeStruct((VOCAB, WIDE), jnp.float32),
        scratch_types=[pltpu.VMEM((HIDDEN, RCH), jnp.float32) for _ in range(2)]
        + [pltpu.VMEM((RCH, WIDE), jnp.float32) for _ in range(2)]
        + [pltpu.VMEM((HIDDEN, HIDDEN), jnp.float32),
           pltpu.VMEM((HIDDEN, WIDE), jnp.float32)]
        + [pltpu.SemaphoreType.DMA for _ in range(5)],
    )(_repack_body)
    t2 = repack(table.T, tail)

    gather = functools.partial(
        pl.kernel,
        mesh=mesh,
        compiler_params=pltpu.CompilerParams(needs_layout_passes=False),
        out_type=jax.ShapeDtypeStruct((SEQ, HIDDEN, BATCH), jnp.float32),
        scratch_types=[pltpu.VMEM((CPW, CHUNK), jnp.int32)]
        + [pltpu.VMEM((CHUNK, WIDE), jnp.float32) for _ in range(NBUF)]
        + [pltpu.VMEM((HIDDEN, CHUNK), jnp.float32) for _ in range(2)]
        + [pltpu.SemaphoreType.DMA for _ in range(NBUF + 2)],
    )(_gather_body)
    outT = gather(input_ids.astype(jnp.int32), t2)
    return jnp.transpose(outT, (0, 2, 1))


# speedup vs baseline: 2.4040x; 2.4040x over previous
import functools

import jax
import jax.numpy as jnp
from jax import lax
from jax.experimental import pallas as pl
from jax.experimental.pallas import tpu as pltpu
from jax.experimental.pallas import tpu_sc as plsc

SEQ = 200
BATCH = 4096
HIDDEN = 64
VOCAB = 1000000
CHUNK = 128
NC = 2
NS = 16
NW = NC * NS
CPW = SEQ
NBUF = 4
LOOK = 2


def _gather_body(idx_hbm, table_hbm, out_hbm, idx_v, *rest):
    gbufs = rest[:NBUF]
    sems = rest[NBUF:2 * NBUF]
    stsems = rest[2 * NBUF:]
    wid = lax.axis_index("s") * NC + lax.axis_index("c")
    col0 = wid * CHUNK

    def out_at(c):
        return out_hbm.at[c, pl.ds(col0, CHUNK), :]

    def gather(c, b):
        pltpu.make_async_copy(
            table_hbm.at[idx_v.at[c]], gbufs[b], sems[b]).start()

    def store(c, b):
        return pltpu.make_async_copy(gbufs[b], out_at(c), stsems[b])

    pltpu.sync_copy(idx_hbm.at[:, pl.ds(col0, CHUNK)], idx_v)

    for c in range(LOOK):
        gather(c, c % NBUF)

    def group(g, carry):
        for b in range(NBUF):
            c = g * NBUF + b
            pb = (b + LOOK) % NBUF

            @pl.when(c + LOOK < CPW)
            def _():
                @pl.when(c >= NBUF - LOOK)
                def _():
                    store(c + LOOK - NBUF, pb).wait()

                gather(c + LOOK, pb)

            pltpu.make_async_copy(
                table_hbm.at[idx_v.at[c]], gbufs[b], sems[b]).wait()
            store(c, b).start()
        return carry

    lax.fori_loop(0, CPW // NBUF, group, 0)

    for c in range(CPW - NBUF, CPW):
        store(c, c % NBUF).wait()


def kernel(input_ids, table):
    mesh = plsc.VectorSubcoreMesh(core_axis_name="c", subcore_axis_name="s")
    run = functools.partial(
        pl.kernel,
        mesh=mesh,
        compiler_params=pltpu.CompilerParams(use_tc_tiling_on_sc=False),
        out_type=jax.ShapeDtypeStruct((SEQ, BATCH, HIDDEN), jnp.float32),
        scratch_types=[pltpu.VMEM((CPW, CHUNK), jnp.int32)]
        + [pltpu.VMEM((CHUNK, HIDDEN), jnp.float32) for _ in range(NBUF)]
        + [pltpu.SemaphoreType.DMA for _ in range(2 * NBUF)],
    )(_gather_body)
    return run(input_ids.astype(jnp.int32), table)
